# baseline (speedup 1.0000x reference)
import functools

import jax
import jax.numpy as jnp
from jax import lax
from jax.experimental import pallas as pl
from jax.experimental.pallas import tpu as pltpu
from jax.experimental.pallas import tpu_sc as plsc

H = 128
N_NODES = 10000
NP = 10240
E = 320000
EP = 327680
NWIN = 128
NBK = NP // NWIN
SBLK = 512
MAXCH = 720
EPAD2 = MAXCH * SBLK
EBLK = 1280
NBLK = 1024
NCORES = 2
NSUB = 16
CHUNK = 128

_PREC = lax.Precision.DEFAULT


def _dot(a, b):
    return lax.dot_general(a, b, (((1,), (0,)), ((), ())),
                           precision=_PREC, preferred_element_type=jnp.float32)


def _dotT(a, b):
    return lax.dot_general(a, b, (((0,), (0,)), ((), ())),
                           precision=_PREC, preferred_element_type=jnp.float32)


def _ln(x, g, b):
    mu = jnp.mean(x, axis=-1, keepdims=True)
    xc = x - mu
    var = jnp.mean(xc * xc, axis=-1, keepdims=True)
    return xc / jnp.sqrt(var + 1e-5) * g + b


def _full(shape):
    return pl.BlockSpec(shape, lambda i: tuple(0 for _ in shape))


def _row_spec(blk):
    return pl.BlockSpec((blk, H), lambda i: (i, 0))


def _node_enc_body(x_ref, emb_ref, w1a_ref, w1b_ref, b1_ref, w2_ref, b2_ref,
                   w3_ref, b3_ref, g_ref, be_ref, o_ref, ob_ref):
    w1_top = _dot(emb_ref[...], w1a_ref[...])
    w1 = jnp.concatenate([w1_top, w1b_ref[...]], axis=0)
    h = jnp.maximum(_dot(x_ref[...], w1) + b1_ref[...], 0.0)
    h = jnp.maximum(_dot(h, w2_ref[...]) + b2_ref[...], 0.0)
    h = _dot(h, w3_ref[...]) + b3_ref[...]
    o = _ln(h, g_ref[...], be_ref[...])
    o_ref[...] = o
    ob_ref[...] = o.astype(jnp.bfloat16)


def _mlp3_body(x_ref, w1_ref, b1_ref, w2_ref, b2_ref, w3_ref, b3_ref,
               g_ref, be_ref, o_ref):
    h = jnp.maximum(_dot(x_ref[...], w1_ref[...]) + b1_ref[...], 0.0)
    h = jnp.maximum(_dot(h, w2_ref[...]) + b2_ref[...], 0.0)
    h = _dot(h, w3_ref[...]) + b3_ref[...]
    o_ref[...] = _ln(h, g_ref[...], be_ref[...])


def _mlp3_noln_body(x_ref, w1_ref, b1_ref, w2_ref, b2_ref, w3_ref, b3_ref,
                    o_ref):
    h = jnp.maximum(_dot(x_ref[...], w1_ref[...]) + b1_ref[...], 0.0)
    h = jnp.maximum(_dot(h, w2_ref[...]) + b2_ref[...], 0.0)
    o_ref[...] = _dot(h, w3_ref[...]) + b3_ref[...]


def _b_body(n_ref, wj_ref, b_ref):
    b_ref[...] = _dot(n_ref[...], wj_ref[...])


def _edge_aggr_body(blk_sref, first_sref, nwin_ref, sdst_ref, gs_ref, e_ref,
                    w1i_ref, w1e_ref, b1_ref, w2_ref, b2_ref, w3_ref, b3_ref,
                    g_ref, be_ref, eo_ref, ag_ref):
    i = pl.program_id(0)
    base = blk_sref[i] * NWIN
    sd = sdst_ref[0]
    ids = lax.broadcasted_iota(jnp.int32, (NWIN, SBLK), 0) + base
    maskT = (ids == jnp.broadcast_to(sd, (NWIN, SBLK))).astype(jnp.bfloat16)
    nwin = nwin_ref[...]
    x_i = _dotT(maskT, nwin)
    x = e_ref[...]
    h = (_dot(x_i, w1i_ref[...]) + gs_ref[...] + _dot(x, w1e_ref[...])
         + b1_ref[...])
    h = jnp.maximum(h, 0.0)
    h = jnp.maximum(_dot(h, w2_ref[...]) + b2_ref[...], 0.0)
    msg = _ln(_dot(h, w3_ref[...]) + b3_ref[...], g_ref[...], be_ref[...])
    eo_ref[...] = x + msg
    m_hi = msg.astype(jnp.bfloat16)
    m_lo = (msg - m_hi.astype(jnp.float32)).astype(jnp.bfloat16)
    part = _dot(maskT, m_hi) + _dot(maskT, m_lo)
    prev = jnp.where(first_sref[i] == 1, 0.0, ag_ref[...])
    ag_ref[...] = prev + part


def _node_upd_body(n_ref, a_ref, wn_ref, wa_ref, b1_ref, w2_ref,
                   b2_ref, w3_ref, b3_ref, g_ref, be_ref, no_ref, nb_ref):
    x = n_ref[...]
    h = jnp.maximum(_dot(x, wn_ref[...]) + _dot(a_ref[...], wa_ref[...])
                    + b1_ref[...], 0.0)
    h = jnp.maximum(_dot(h, w2_ref[...]) + b2_ref[...], 0.0)
    o = x + _ln(_dot(h, w3_ref[...]) + b3_ref[...], g_ref[...], be_ref[...])
    no_ref[...] = o
    nb_ref[...] = o.astype(jnp.bfloat16)


def _wspecs(shapes):
    return [_full(s) for s in shapes]


def _node_encode(feat, emb_p, w1a, w1b_p, b1, w2, b2, w3, b3, g, be):
    return pl.pallas_call(
        _node_enc_body,
        grid=(NP // NBLK,),
        in_specs=[_row_spec(NBLK)] + _wspecs([
            (16, 16), (16, H), (H - 16, H), (1, H), (H, H), (1, H),
            (H, H), (1, H), (1, H), (1, H)]),
        out_specs=[_row_spec(NBLK), _row_spec(NBLK)],
        out_shape=[jax.ShapeDtypeStruct((NP, H), jnp.float32),
                   jax.ShapeDtypeStruct((NP, H), jnp.bfloat16)],
    )(feat, emb_p, w1a, w1b_p, b1, w2, b2, w3, b3, g, be)


def _edge_encode(feat, w1, b1, w2, b2, w3, b3, g, be):
    return pl.pallas_call(
        _mlp3_body,
        grid=(EPAD2 // EBLK,),
        in_specs=[_row_spec(EBLK)] + _wspecs([
            (H, H), (1, H), (H, H), (1, H), (H, H), (1, H), (1, H), (1, H)]),
        out_specs=_row_spec(EBLK),
        out_shape=jax.ShapeDtypeStruct((EPAD2, H), jnp.float32),
    )(feat, w1, b1, w2, b2, w3, b3, g, be)


def _b_project(node, wj):
    return pl.pallas_call(
        _b_body,
        grid=(NP // NBLK,),
        in_specs=[_row_spec(NBLK)] + _wspecs([(H, H)]),
        out_specs=_row_spec(NBLK),
        out_shape=jax.ShapeDtypeStruct((NP, H), jnp.float32),
    )(node, wj)


def _edge_aggr(blkid, first, node_bf, sdst3d, gs, e,
               w1i, w1e, b1, w2, b2, w3, b3, g, be):
    grid_spec = pltpu.PrefetchScalarGridSpec(
        num_scalar_prefetch=2,
        grid=(MAXCH,),
        in_specs=[
            pl.BlockSpec((NWIN, H), lambda i, bk, fs: (bk[i], 0)),
            pl.BlockSpec((1, 1, SBLK), lambda i, bk, fs: (i, 0, 0)),
            pl.BlockSpec((SBLK, H), lambda i, bk, fs: (i, 0)),
            pl.BlockSpec((SBLK, H), lambda i, bk, fs: (i, 0)),
        ] + [pl.BlockSpec(s, lambda i, bk, fs: tuple(0 for _ in s))
             for s in [(H, H), (H, H), (1, H), (H, H), (1, H), (H, H),
                       (1, H), (1, H), (1, H)]],
        out_specs=[
            pl.BlockSpec((SBLK, H), lambda i, bk, fs: (i, 0)),
            pl.BlockSpec((NWIN, H), lambda i, bk, fs: (bk[i], 0)),
        ],
    )
    return pl.pallas_call(
        _edge_aggr_body,
        grid_spec=grid_spec,
        out_shape=[jax.ShapeDtypeStruct((EPAD2, H), jnp.float32),
                   jax.ShapeDtypeStruct((NP, H), jnp.float32)],
    )(blkid, first, node_bf, sdst3d, gs, e,
      w1i, w1e, b1, w2, b2, w3, b3, g, be)


def _node_update(node, aggr, wn, wa, b1, w2, b2, w3, b3, g, be):
    return pl.pallas_call(
        _node_upd_body,
        grid=(NP // NBLK,),
        in_specs=[_row_spec(NBLK)] * 2 + _wspecs([
            (H, H), (H, H), (1, H), (H, H), (1, H), (H, H), (1, H),
            (1, H), (1, H)]),
        out_specs=[_row_spec(NBLK), _row_spec(NBLK)],
        out_shape=[jax.ShapeDtypeStruct((NP, H), jnp.float32),
                   jax.ShapeDtypeStruct((NP, H), jnp.bfloat16)],
    )(node, aggr, wn, wa, b1, w2, b2, w3, b3, g, be)


def _decode(node, w1, b1, w2, b2, w3, b3):
    return pl.pallas_call(
        _mlp3_noln_body,
        grid=(NP // NBLK,),
        in_specs=[_row_spec(NBLK)] + _wspecs([
            (H, H), (1, H), (H, H), (1, H), (H, H), (1, H)]),
        out_specs=_row_spec(NBLK),
        out_shape=jax.ShapeDtypeStruct((NP, H), jnp.float32),
    )(node, w1, b1, w2, b2, w3, b3)


def _sc_mesh():
    return plsc.VectorSubcoreMesh(core_axis_name="core",
                                  subcore_axis_name="subcore")


def _sc_gather1(table, idx2d):
    nchunks = idx2d.shape[0]
    nrows = nchunks * CHUNK
    out_t = jax.ShapeDtypeStruct((nrows, H), jnp.float32)

    @functools.partial(pl.kernel, out_type=out_t, mesh=_sc_mesh(),
                       scratch_types=[pltpu.SemaphoreType.DMA,
                                      pltpu.SemaphoreType.DMA])
    def k(t_hbm, i_hbm, o_hbm, s0, s1):
        def body(i_v, o_v):
            c0 = pltpu.async_copy(t_hbm.at[i_v.at[0]],
                                  o_v.at[pl.ds(0, CHUNK)], s0)
            c1 = pltpu.async_copy(t_hbm.at[i_v.at[1]],
                                  o_v.at[pl.ds(CHUNK, CHUNK)], s1)
            c0.wait()
            c1.wait()

        pltpu.emit_pipeline(
            body,
            grid=(nchunks // 2,),
            in_specs=[pl.BlockSpec((2, CHUNK), lambda i: (i, 0))],
            out_specs=[pl.BlockSpec((2 * CHUNK, H), lambda i: (i, 0))],
            core_axis_name=("core", "subcore"),
            dimension_semantics=(pltpu.PARALLEL,),
        )(i_hbm, o_hbm)

    return k(table, idx2d)


def kernel(x, pos, edge_index, edge_attr, params):
    f32 = jnp.float32
    i32 = jnp.int32
    x = x.astype(i32)
    ei = edge_index.astype(i32)
    src, dst = ei[0], ei[1]

    dstb = dst >> 7
    order = jnp.argsort(dstb)
    sdstb = dstb[order]
    starts = jnp.searchsorted(sdstb, jnp.arange(NBK, dtype=i32))
    ends = jnp.searchsorted(sdstb, jnp.arange(1, NBK + 1, dtype=i32))
    counts = (ends - starts).astype(i32)
    cb = jnp.maximum(1, (counts + SBLK - 1) // SBLK)
    cstart = jnp.concatenate([jnp.zeros((1,), i32), jnp.cumsum(cb)])
    total = cstart[NBK]
    ii = jnp.arange(MAXCH, dtype=i32)
    cbk = jnp.clip(jnp.searchsorted(cstart, ii, side="right") - 1, 0, NBK - 1)
    blkid = jnp.where(ii < total, cbk, NBK - 1).astype(i32)
    first = jnp.where((ii < total) & (ii == cstart[cbk]), 1, 0).astype(i32)
    slot = (SBLK * cstart[sdstb]
            + (jnp.arange(E, dtype=i32) - starts[sdstb])).astype(i32)
    inv = jnp.full((EPAD2,), E, i32).at[slot].set(order.astype(i32))
    sdst_p = jnp.full((EPAD2,), -1, i32).at[slot].set(dst[order])
    ssrc_p = jnp.full((EPAD2,), N_NODES, i32).at[slot].set(src[order])
    sdst3d = sdst_p.reshape(MAXCH, 1, SBLK)
    ssrc2d = ssrc_p.reshape(EPAD2 // CHUNK, CHUNK)
    inv2d = inv.reshape(EPAD2 // CHUNK, CHUNK)

    def b2d(b):
        return b.reshape(1, -1)

    oh = (x[:, None] == jnp.arange(9, dtype=i32)[None, :]).astype(f32)
    feat = jnp.concatenate([oh, jnp.zeros((N_NODES, 7), f32), pos], axis=1)
    feat = jnp.pad(feat, ((0, NP - N_NODES), (0, H - feat.shape[1])))

    pni = params["node_in"]
    (w1, b1), (w2, b2), (w3, b3) = pni["lin"]
    g, be = pni["ln"]
    emb_p = jnp.pad(params["embed"], ((0, 7), (0, 0)))
    w1a = w1[:16]
    w1b_p = jnp.pad(w1[16:30], ((0, (H - 16) - 14), (0, 0)))
    node, node_bf = _node_encode(feat, emb_p, w1a, w1b_p, b2d(b1), w2, b2d(b2),
                                 w3, b2d(b3), b2d(g), b2d(be))

    pei = params["edge_in"]
    (w1, b1), (w2, b2), (w3, b3) = pei["lin"]
    g, be = pei["ln"]
    ef = jnp.pad(edge_attr, ((0, EP - E), (0, H - edge_attr.shape[1])))
    ef_p = _sc_gather1(ef, inv2d)
    w1_p = jnp.pad(w1, ((0, H - w1.shape[0]), (0, 0)))
    edge = _edge_encode(ef_p, w1_p, b2d(b1), w2, b2d(b2), w3, b2d(b3),
                        b2d(g), b2d(be))

    for layer in params["mp"]:
        (we1, eb1), (we2, eb2), (we3, eb3) = layer["lin_edge"]["lin"]
        eg, ebe = layer["lin_edge"]["ln"]
        w1i, w1j, w1e = we1[:H], we1[H:2 * H], we1[2 * H:]
        bproj = _b_project(node, w1j)
        gs = _sc_gather1(bproj, ssrc2d)
        edge, aggr = _edge_aggr(blkid, first, node_bf, sdst3d, gs, edge,
                                w1i, w1e, b2d(eb1), we2, b2d(eb2),
                                we3, b2d(eb3), b2d(eg), b2d(ebe))
        (wn1, nb1), (wn2, nb2), (wn3, nb3) = layer["lin_node"]["lin"]
        ng, nbe = layer["lin_node"]["ln"]
        node, node_bf = _node_update(node, aggr, wn1[:H], wn1[H:],
                                     b2d(nb1), wn2, b2d(nb2), wn3, b2d(nb3),
                                     b2d(ng), b2d(nbe))

    pno = params["node_out"]
    (w1, b1), (w2, b2), (w3, b3) = pno["lin"]
    w3_p = jnp.pad(w3, ((0, 0), (0, H - w3.shape[1])))
    b3_p = jnp.pad(b3, (0, H - b3.shape[0]))
    out = _decode(node, w1, b2d(b1), w2, b2d(b2), w3_p, b2d(b3_p))
    return out[:N_NODES, :2]


# speedup vs baseline: 3.4395x; 3.4395x over previous
import functools

import jax
import jax.numpy as jnp
from jax import lax
from jax.experimental import pallas as pl
from jax.experimental.pallas import tpu as pltpu
from jax.experimental.pallas import tpu_sc as plsc

H = 128
N_NODES = 10000
NP = 10240
E = 320000
EP = 327680
EBLK = 1280
NBLK = 1024
NCORES = 2
NSUB = 16
CHUNK = 128
EPH = EP // 2
NCHUNKS_H = EPH // CHUNK
CHUNKS_PER_TILE_H = NCHUNKS_H // (NCORES * NSUB)
ACC_ROWS_PER_TILE = NP // NSUB

_PREC = lax.Precision.DEFAULT


def _dot(a, b):
    return lax.dot_general(a, b, (((1,), (0,)), ((), ())),
                           precision=_PREC, preferred_element_type=jnp.float32)


def _ln(x, g, b):
    mu = jnp.mean(x, axis=-1, keepdims=True)
    xc = x - mu
    var = jnp.mean(xc * xc, axis=-1, keepdims=True)
    return xc / jnp.sqrt(var + 1e-5) * g + b


def _full(shape):
    return pl.BlockSpec(shape, lambda i: tuple(0 for _ in shape))


def _row_spec(blk):
    return pl.BlockSpec((blk, H), lambda i: (i, 0))


def _node_enc_body(x_ref, emb_ref, w1a_ref, w1b_ref, b1_ref, w2_ref, b2_ref,
                   w3_ref, b3_ref, g_ref, be_ref, o_ref, ob_ref):
    w1_top = _dot(emb_ref[...], w1a_ref[...])
    w1 = jnp.concatenate([w1_top, w1b_ref[...]], axis=0)
    h = jnp.maximum(_dot(x_ref[...], w1) + b1_ref[...], 0.0)
    h = jnp.maximum(_dot(h, w2_ref[...]) + b2_ref[...], 0.0)
    h = _dot(h, w3_ref[...]) + b3_ref[...]
    o = _ln(h, g_ref[...], be_ref[...])
    o_ref[...] = o
    ob_ref[...] = o.astype(jnp.bfloat16)


def _mlp3_body(x_ref, w1_ref, b1_ref, w2_ref, b2_ref, w3_ref, b3_ref,
               g_ref, be_ref, o_ref):
    h = jnp.maximum(_dot(x_ref[...], w1_ref[...]) + b1_ref[...], 0.0)
    h = jnp.maximum(_dot(h, w2_ref[...]) + b2_ref[...], 0.0)
    h = _dot(h, w3_ref[...]) + b3_ref[...]
    o_ref[...] = _ln(h, g_ref[...], be_ref[...])


def _mlp3_noln_body(x_ref, w1_ref, b1_ref, w2_ref, b2_ref, w3_ref, b3_ref,
                    o_ref):
    h = jnp.maximum(_dot(x_ref[...], w1_ref[...]) + b1_ref[...], 0.0)
    h = jnp.maximum(_dot(h, w2_ref[...]) + b2_ref[...], 0.0)
    o_ref[...] = _dot(h, w3_ref[...]) + b3_ref[...]


def _ab_body(n_ref, wi_ref, wj_ref, a_ref, b_ref):
    a_ref[...] = _dot(n_ref[...], wi_ref[...])
    b_ref[...] = _dot(n_ref[...], wj_ref[...])


def _edge_body(gd_ref, gs_ref, e_ref, w1e_ref, b1_ref, w2_ref, b2_ref,
               w3_ref, b3_ref, g_ref, be_ref, eo_ref, mo_ref):
    x = e_ref[...]
    h = gd_ref[...] + gs_ref[...] + _dot(x, w1e_ref[...]) + b1_ref[...]
    h = jnp.maximum(h, 0.0)
    h = jnp.maximum(_dot(h, w2_ref[...]) + b2_ref[...], 0.0)
    m = _ln(_dot(h, w3_ref[...]) + b3_ref[...], g_ref[...], be_ref[...])
    mo_ref[...] = m
    eo_ref[...] = x + m


def _node_upd_body(n_ref, a0_ref, a1_ref, a2_ref, a3_ref, wn_ref, wa_ref,
                   b1_ref, w2_ref, b2_ref, w3_ref, b3_ref, g_ref, be_ref,
                   no_ref, nb_ref):
    x = n_ref[...]
    acc = (a0_ref[...] + a1_ref[...]) + (a2_ref[...] + a3_ref[...])
    h = jnp.maximum(_dot(x, wn_ref[...]) + _dot(acc, wa_ref[...]) + b1_ref[...], 0.0)
    h = jnp.maximum(_dot(h, w2_ref[...]) + b2_ref[...], 0.0)
    o = x + _ln(_dot(h, w3_ref[...]) + b3_ref[...], g_ref[...], be_ref[...])
    no_ref[...] = o
    nb_ref[...] = o.astype(jnp.bfloat16)


def _wspecs(shapes):
    return [_full(s) for s in shapes]


def _node_encode(feat, emb_p, w1a, w1b_p, b1, w2, b2, w3, b3, g, be):
    return pl.pallas_call(
        _node_enc_body,
        grid=(NP // NBLK,),
        in_specs=[_row_spec(NBLK)] + _wspecs([
            (16, 16), (16, H), (H - 16, H), (1, H), (H, H), (1, H),
            (H, H), (1, H), (1, H), (1, H)]),
        out_specs=[_row_spec(NBLK), _row_spec(NBLK)],
        out_shape=[jax.ShapeDtypeStruct((NP, H), jnp.float32),
                   jax.ShapeDtypeStruct((NP, H), jnp.bfloat16)],
    )(feat, emb_p, w1a, w1b_p, b1, w2, b2, w3, b3, g, be)


def _edge_encode_half(feat, half, w1, b1, w2, b2, w3, b3, g, be):
    nblocks = EPH // EBLK
    return pl.pallas_call(
        _mlp3_body,
        grid=(nblocks,),
        in_specs=[pl.BlockSpec((EBLK, H), lambda i: (i + half * nblocks, 0))]
        + _wspecs([
            (H, H), (1, H), (H, H), (1, H), (H, H), (1, H), (1, H), (1, H)]),
        out_specs=_row_spec(EBLK),
        out_shape=jax.ShapeDtypeStruct((EPH, H), jnp.float32),
    )(feat, w1, b1, w2, b2, w3, b3, g, be)


def _ab_project(node, wi, wj):
    return pl.pallas_call(
        _ab_body,
        grid=(NP // NBLK,),
        in_specs=[_row_spec(NBLK)] + _wspecs([(H, H), (H, H)]),
        out_specs=[_row_spec(NBLK), _row_spec(NBLK)],
        out_shape=[jax.ShapeDtypeStruct((NP, H), jnp.float32)] * 2,
    )(node, wi, wj)


def _edge_mlp(gd, gs, e, w1e, b1, w2, b2, w3, b3, g, be):
    return pl.pallas_call(
        _edge_body,
        grid=(EPH // EBLK,),
        in_specs=[_row_spec(EBLK)] * 3 + _wspecs([
            (H, H), (1, H), (H, H), (1, H), (H, H), (1, H), (1, H), (1, H)]),
        out_specs=[_row_spec(EBLK), _row_spec(EBLK)],
        out_shape=[jax.ShapeDtypeStruct((EPH, H), jnp.float32)] * 2,
    )(gd, gs, e, w1e, b1, w2, b2, w3, b3, g, be)


def _node_update(node, acc0, acc1, acc2, acc3, wn, wa, b1, w2, b2, w3, b3,
                 g, be):
    return pl.pallas_call(
        _node_upd_body,
        grid=(NP // NBLK,),
        in_specs=[_row_spec(NBLK)] * 5 + _wspecs([
            (H, H), (H, H), (1, H), (H, H), (1, H), (H, H), (1, H),
            (1, H), (1, H)]),
        out_specs=[_row_spec(NBLK), _row_spec(NBLK)],
        out_shape=[jax.ShapeDtypeStruct((NP, H), jnp.float32),
                   jax.ShapeDtypeStruct((NP, H), jnp.bfloat16)],
    )(node, acc0, acc1, acc2, acc3, wn, wa, b1, w2, b2, w3, b3, g, be)


def _decode(node, w1, b1, w2, b2, w3, b3):
    return pl.pallas_call(
        _mlp3_noln_body,
        grid=(NP // NBLK,),
        in_specs=[_row_spec(NBLK)] + _wspecs([
            (H, H), (1, H), (H, H), (1, H), (H, H), (1, H)]),
        out_specs=_row_spec(NBLK),
        out_shape=jax.ShapeDtypeStruct((NP, H), jnp.float32),
    )(node, w1, b1, w2, b2, w3, b3)


def _sc_mesh():
    return plsc.VectorSubcoreMesh(core_axis_name="core",
                                  subcore_axis_name="subcore")


def _sc_gather(a, b, dst2d, src2d):
    out_t = (jax.ShapeDtypeStruct((EPH, H), jnp.float32),
             jax.ShapeDtypeStruct((EPH, H), jnp.float32))

    @functools.partial(pl.kernel, out_type=out_t, mesh=_sc_mesh(),
                       scratch_types=[pltpu.SemaphoreType.DMA,
                                      pltpu.SemaphoreType.DMA])
    def k(a_hbm, b_hbm, d_hbm, s_hbm, gd_hbm, gs_hbm, sg0, sg1):
        def body(d_v, s_v, gd_v, gs_v):
            c0 = pltpu.async_copy(a_hbm.at[d_v.at[0]], gd_v, sg0)
            c1 = pltpu.async_copy(b_hbm.at[s_v.at[0]], gs_v, sg1)
            c0.wait()
            c1.wait()

        pltpu.emit_pipeline(
            body,
            grid=(NCHUNKS_H,),
            in_specs=[pl.BlockSpec((1, CHUNK), lambda i: (i, 0)),
                      pl.BlockSpec((1, CHUNK), lambda i: (i, 0))],
            out_specs=[pl.BlockSpec((CHUNK, H), lambda i: (i, 0)),
                       pl.BlockSpec((CHUNK, H), lambda i: (i, 0))],
            core_axis_name=("core", "subcore"),
            dimension_semantics=(pltpu.PARALLEL,),
        )(d_hbm, s_hbm, gd_hbm, gs_hbm)

    return k(a, b, dst2d, src2d)


def _sc_scatter(msg, dst2d, zeros):
    @functools.partial(
        pl.kernel,
        out_type=jax.ShapeDtypeStruct((NCORES, NP, H), jnp.float32),
        mesh=_sc_mesh(),
        scratch_types=[
            pltpu.VMEM((2, CHUNK, H), jnp.float32),
            pltpu.VMEM((2, CHUNK), jnp.int32),
            pltpu.VMEM_SHARED((NP, H), jnp.float32),
            pltpu.SemaphoreType.DMA,
            pltpu.SemaphoreType.DMA,
        ],
    )
    def k(m_hbm, d_hbm, z_hbm, o_hbm, m_v, i_v, acc, sem0, sem1):
        cid = lax.axis_index("core")
        sid = lax.axis_index("subcore")

        @pl.loop(0, ACC_ROWS_PER_TILE // CHUNK)
        def _zero(j):
            pltpu.sync_copy(
                z_hbm, acc.at[pl.ds(sid * ACC_ROWS_PER_TILE + j * CHUNK, CHUNK)])

        plsc.subcore_barrier()

        base = (cid * NSUB + sid) * CHUNKS_PER_TILE_H
        sems = (sem0, sem1)

        def load(c, slot, sem):
            pltpu.async_copy(m_hbm.at[pl.ds(c * CHUNK, CHUNK)],
                             m_v.at[slot], sem)
            pltpu.async_copy(d_hbm.at[pl.ds(c, 1)],
                             i_v.at[pl.ds(slot, 1)], sem)

        def drain_add(c, slot, sem):
            pltpu.make_async_copy(m_hbm.at[pl.ds(c * CHUNK, CHUNK)],
                                  m_v.at[slot], sem).wait()
            pltpu.make_async_copy(d_hbm.at[pl.ds(c, 1)],
                                  i_v.at[pl.ds(slot, 1)], sem).wait()
            pltpu.sync_copy(m_v.at[slot], acc.at[i_v.at[slot]], add=True)

        load(base, 0, sems[0])

        @pl.loop(0, CHUNKS_PER_TILE_H // 2)
        def _scat(t):
            c = base + 2 * t
            load(c + 1, 1, sems[1])
            drain_add(c, 0, sems[0])

            @pl.when(t < CHUNKS_PER_TILE_H // 2 - 1)
            def _():
                load(c + 2, 0, sems[0])

            drain_add(c + 1, 1, sems[1])

        plsc.subcore_barrier()

        @pl.loop(0, ACC_ROWS_PER_TILE // CHUNK)
        def _drain(j):
            r = sid * ACC_ROWS_PER_TILE + j * CHUNK
            pltpu.sync_copy(acc.at[pl.ds(r, CHUNK)],
                            o_hbm.at[cid, pl.ds(r, CHUNK)])

    return k(msg, dst2d, zeros)


def kernel(x, pos, edge_index, edge_attr, params):
    f32 = jnp.float32
    x = x.astype(jnp.int32)
    ei = edge_index.astype(jnp.int32)
    src, dst = ei[0], ei[1]
    padidx = jnp.full((EP - E,), N_NODES, jnp.int32)
    dst2d = jnp.concatenate([dst, padidx]).reshape(2, NCHUNKS_H, CHUNK)
    src2d = jnp.concatenate([src, padidx]).reshape(2, NCHUNKS_H, CHUNK)
    dst_h = (dst2d[0], dst2d[1])
    src_h = (src2d[0], src2d[1])

    def b2d(b):
        return b.reshape(1, -1)

    def pad_lane(a, n):
        return jnp.pad(a, ((0, 0), (0, n - a.shape[1])))

    oh = (x[:, None] == jnp.arange(9, dtype=jnp.int32)[None, :]).astype(f32)
    feat = jnp.concatenate([oh, jnp.zeros((N_NODES, 7), f32), pos], axis=1)
    feat = jnp.pad(feat, ((0, NP - N_NODES), (0, H - feat.shape[1])))

    pni = params["node_in"]
    (w1, b1), (w2, b2), (w3, b3) = pni["lin"]
    g, be = pni["ln"]
    emb_p = jnp.pad(params["embed"], ((0, 7), (0, 0)))
    w1a = w1[:16]
    w1b_p = jnp.pad(w1[16:30], ((0, (H - 16) - 14), (0, 0)))
    node, node_bf = _node_encode(feat, emb_p, w1a, w1b_p, b2d(b1), w2, b2d(b2),
                                 w3, b2d(b3), b2d(g), b2d(be))

    pei = params["edge_in"]
    (w1, b1), (w2, b2), (w3, b3) = pei["lin"]
    g, be = pei["ln"]
    ef = jnp.pad(edge_attr, ((0, EP - E), (0, H - edge_attr.shape[1])))
    w1_p = jnp.pad(w1, ((0, H - w1.shape[0]), (0, 0)))
    e_h = [_edge_encode_half(ef, hf, w1_p, b2d(b1), w2, b2d(b2), w3, b2d(b3),
                             b2d(g), b2d(be)) for hf in (0, 1)]

    zeros = jnp.zeros((CHUNK, H), f32)

    for layer in params["mp"]:
        (we1, eb1), (we2, eb2), (we3, eb3) = layer["lin_edge"]["lin"]
        eg, ebe = layer["lin_edge"]["ln"]
        w1i, w1j, w1e = we1[:H], we1[H:2 * H], we1[2 * H:]
        a, b = _ab_project(node, w1i, w1j)
        ew = (w1e, b2d(eb1), we2, b2d(eb2), we3, b2d(eb3), b2d(eg), b2d(ebe))
        g0d, g0s = _sc_gather(a, b, dst_h[0], src_h[0])
        g1d, g1s = _sc_gather(a, b, dst_h[1], src_h[1])
        e0, m0 = _edge_mlp(g0d, g0s, e_h[0], *ew)
        acc0 = _sc_scatter(m0, dst_h[0], zeros)
        e1, m1 = _edge_mlp(g1d, g1s, e_h[1], *ew)
        acc1 = _sc_scatter(m1, dst_h[1], zeros)
        e_h = [e0, e1]
        (wn1, nb1), (wn2, nb2), (wn3, nb3) = layer["lin_node"]["lin"]
        ng, nbe = layer["lin_node"]["ln"]
        node, node_bf = _node_update(node, acc0[0], acc0[1], acc1[0], acc1[1],
                                     wn1[:H], wn1[H:],
                                     b2d(nb1), wn2, b2d(nb2), wn3, b2d(nb3),
                                     b2d(ng), b2d(nbe))

    pno = params["node_out"]
    (w1, b1), (w2, b2), (w3, b3) = pno["lin"]
    w3_p = pad_lane(w3, H)
    b3_p = jnp.pad(b3, (0, H - b3.shape[0]))
    out = _decode(node, w1, b2d(b1), w2, b2d(b2), w3_p, b2d(b3_p))
    return out[:N_NODES, :2]


# speedup vs baseline: 8.6004x; 2.5005x over previous
import functools

import jax
import jax.numpy as jnp
from jax import lax
from jax.experimental import pallas as pl
from jax.experimental.pallas import tpu as pltpu
from jax.experimental.pallas import tpu_sc as plsc

H = 128
N_NODES = 10000
NP = 10240
E = 320000
EP = 327680
EBLK = 1280
NBLK = 1024
NCORES = 2
NSUB = 16
CHUNK = 128
EPH = EP // 2
NCHUNKS_H = EPH // CHUNK
CHUNKS_PER_TILE_H = NCHUNKS_H // (NCORES * NSUB)
ACC_ROWS_PER_TILE = NP // NSUB

_PREC = lax.Precision.DEFAULT


def _dot(a, b):
    return lax.dot_general(a, b, (((1,), (0,)), ((), ())),
                           precision=_PREC, preferred_element_type=jnp.float32)


def _ln(x, g, b):
    mu = jnp.mean(x, axis=-1, keepdims=True)
    xc = x - mu
    var = jnp.mean(xc * xc, axis=-1, keepdims=True)
    return xc / jnp.sqrt(var + 1e-5) * g + b


def _full(shape):
    return pl.BlockSpec(shape, lambda i: tuple(0 for _ in shape))


def _row_spec(blk):
    return pl.BlockSpec((blk, H), lambda i: (i, 0))


def _node_enc_body(x_ref, emb_ref, w1a_ref, w1b_ref, b1_ref, w2_ref, b2_ref,
                   w3_ref, b3_ref, g_ref, be_ref, o_ref):
    w1_top = _dot(emb_ref[...], w1a_ref[...])
    x = x_ref[...]
    h_emb = lax.dot_general(x[:, :16], w1_top, (((1,), (0,)), ((), ())),
                            precision=lax.Precision.HIGHEST,
                            preferred_element_type=jnp.float32)
    h = jnp.maximum(h_emb + _dot(x[:, 16:], w1b_ref[...]) + b1_ref[...], 0.0)
    h = jnp.maximum(_dot(h, w2_ref[...]) + b2_ref[...], 0.0)
    h = _dot(h, w3_ref[...]) + b3_ref[...]
    o_ref[...] = _ln(h, g_ref[...], be_ref[...])


def _mlp3_body(x_ref, w1_ref, b1_ref, w2_ref, b2_ref, w3_ref, b3_ref,
               g_ref, be_ref, o_ref):
    h = jnp.maximum(_dot(x_ref[...], w1_ref[...]) + b1_ref[...], 0.0)
    h = jnp.maximum(_dot(h, w2_ref[...]) + b2_ref[...], 0.0)
    h = _dot(h, w3_ref[...]) + b3_ref[...]
    o_ref[...] = _ln(h, g_ref[...], be_ref[...])


def _mlp3_noln_body(x_ref, w1_ref, b1_ref, w2_ref, b2_ref, w3_ref, b3_ref,
                    o_ref):
    h = jnp.maximum(_dot(x_ref[...], w1_ref[...]) + b1_ref[...], 0.0)
    h = jnp.maximum(_dot(h, w2_ref[...]) + b2_ref[...], 0.0)
    o_ref[...] = _dot(h, w3_ref[...]) + b3_ref[...]


def _edge_body(gd_ref, gs_ref, e_ref, w1_ref, b1_ref, w2_ref, b2_ref,
               w3_ref, b3_ref, g_ref, be_ref, eo_ref, mo_ref):
    x = e_ref[...]
    xc = jnp.concatenate([gd_ref[...], gs_ref[...], x], axis=1)
    h = jnp.maximum(_dot(xc, w1_ref[...]) + b1_ref[...], 0.0)
    h = jnp.maximum(_dot(h, w2_ref[...]) + b2_ref[...], 0.0)
    m = _ln(_dot(h, w3_ref[...]) + b3_ref[...], g_ref[...], be_ref[...])
    mo_ref[...] = m
    eo_ref[...] = x + m


def _node_upd_body(n_ref, a0_ref, a1_ref, a2_ref, a3_ref, w1_ref,
                   b1_ref, w2_ref, b2_ref, w3_ref, b3_ref, g_ref, be_ref,
                   no_ref):
    x = n_ref[...]
    acc = (a0_ref[...] + a1_ref[...]) + (a2_ref[...] + a3_ref[...])
    xc = jnp.concatenate([x, acc], axis=1)
    h = jnp.maximum(_dot(xc, w1_ref[...]) + b1_ref[...], 0.0)
    h = jnp.maximum(_dot(h, w2_ref[...]) + b2_ref[...], 0.0)
    no_ref[...] = x + _ln(_dot(h, w3_ref[...]) + b3_ref[...],
                          g_ref[...], be_ref[...])


def _wspecs(shapes):
    return [_full(s) for s in shapes]


def _node_encode(feat, emb_p, w1a, w1b_p, b1, w2, b2, w3, b3, g, be):
    return pl.pallas_call(
        _node_enc_body,
        grid=(NP // NBLK,),
        in_specs=[_row_spec(NBLK)] + _wspecs([
            (16, 16), (16, H), (H - 16, H), (1, H), (H, H), (1, H),
            (H, H), (1, H), (1, H), (1, H)]),
        out_specs=_row_spec(NBLK),
        out_shape=jax.ShapeDtypeStruct((NP, H), jnp.float32),
    )(feat, emb_p, w1a, w1b_p, b1, w2, b2, w3, b3, g, be)


def _edge_encode_half(feat, half, w1, b1, w2, b2, w3, b3, g, be):
    nblocks = EPH // EBLK
    return pl.pallas_call(
        _mlp3_body,
        grid=(nblocks,),
        in_specs=[pl.BlockSpec((EBLK, H), lambda i: (i + half * nblocks, 0))]
        + _wspecs([
            (H, H), (1, H), (H, H), (1, H), (H, H), (1, H), (1, H), (1, H)]),
        out_specs=_row_spec(EBLK),
        out_shape=jax.ShapeDtypeStruct((EPH, H), jnp.float32),
    )(feat, w1, b1, w2, b2, w3, b3, g, be)


def _edge_mlp(gd, gs, e, w1, b1, w2, b2, w3, b3, g, be):
    return pl.pallas_call(
        _edge_body,
        grid=(EPH // EBLK,),
        in_specs=[_row_spec(EBLK)] * 3 + _wspecs([
            (3 * H, H), (1, H), (H, H), (1, H), (H, H), (1, H), (1, H),
            (1, H)]),
        out_specs=[_row_spec(EBLK), _row_spec(EBLK)],
        out_shape=[jax.ShapeDtypeStruct((EPH, H), jnp.float32)] * 2,
    )(gd, gs, e, w1, b1, w2, b2, w3, b3, g, be)


def _node_update(node, acc0, acc1, acc2, acc3, w1, b1, w2, b2, w3, b3, g, be):
    return pl.pallas_call(
        _node_upd_body,
        grid=(NP // NBLK,),
        in_specs=[_row_spec(NBLK)] * 5 + _wspecs([
            (2 * H, H), (1, H), (H, H), (1, H), (H, H), (1, H),
            (1, H), (1, H)]),
        out_specs=_row_spec(NBLK),
        out_shape=jax.ShapeDtypeStruct((NP, H), jnp.float32),
    )(node, acc0, acc1, acc2, acc3, w1, b1, w2, b2, w3, b3, g, be)


def _decode(node, w1, b1, w2, b2, w3, b3):
    return pl.pallas_call(
        _mlp3_noln_body,
        grid=(NP // NBLK,),
        in_specs=[_row_spec(NBLK)] + _wspecs([
            (H, H), (1, H), (H, H), (1, H), (H, H), (1, H)]),
        out_specs=_row_spec(NBLK),
        out_shape=jax.ShapeDtypeStruct((NP, H), jnp.float32),
    )(node, w1, b1, w2, b2, w3, b3)


def _sc_mesh():
    return plsc.VectorSubcoreMesh(core_axis_name="core",
                                  subcore_axis_name="subcore")


def _sc_gather(a, b, dst2d, src2d):
    out_t = (jax.ShapeDtypeStruct((EPH, H), jnp.float32),
             jax.ShapeDtypeStruct((EPH, H), jnp.float32))

    @functools.partial(pl.kernel, out_type=out_t, mesh=_sc_mesh(),
                       scratch_types=[pltpu.SemaphoreType.DMA,
                                      pltpu.SemaphoreType.DMA])
    def k(a_hbm, b_hbm, d_hbm, s_hbm, gd_hbm, gs_hbm, sg0, sg1):
        def body(d_v, s_v, gd_v, gs_v):
            c0 = pltpu.async_copy(a_hbm.at[d_v.at[0]], gd_v, sg0)
            c1 = pltpu.async_copy(b_hbm.at[s_v.at[0]], gs_v, sg1)
            c0.wait()
            c1.wait()

        pltpu.emit_pipeline(
            body,
            grid=(NCHUNKS_H,),
            in_specs=[pl.BlockSpec((1, CHUNK), lambda i: (i, 0)),
                      pl.BlockSpec((1, CHUNK), lambda i: (i, 0))],
            out_specs=[pl.BlockSpec((CHUNK, H), lambda i: (i, 0)),
                       pl.BlockSpec((CHUNK, H), lambda i: (i, 0))],
            core_axis_name=("core", "subcore"),
            dimension_semantics=(pltpu.PARALLEL,),
        )(d_hbm, s_hbm, gd_hbm, gs_hbm)

    return k(a, b, dst2d, src2d)


def _sc_scatter(msg, dst2d):
    @functools.partial(
        pl.kernel,
        out_type=jax.ShapeDtypeStruct((NCORES, NP, H), jnp.float32),
        mesh=_sc_mesh(),
        scratch_types=[
            pltpu.VMEM((2, CHUNK, H), jnp.float32),
            pltpu.VMEM((2, CHUNK), jnp.int32),
            pltpu.VMEM((64, H), jnp.float32),
            pltpu.VMEM_SHARED((NP, H), jnp.float32),
            pltpu.SemaphoreType.DMA,
            pltpu.SemaphoreType.DMA,
        ],
    )
    def k(m_hbm, d_hbm, o_hbm, m_v, i_v, z_v, acc, sem0, sem1):
        cid = lax.axis_index("core")
        sid = lax.axis_index("subcore")

        zv16 = jnp.zeros((16,), jnp.float32)

        @pl.loop(0, 64)
        def _zrow(r):
            @pl.loop(0, H // 16)
            def _zlane(l):
                z_v[r, pl.ds(l * 16, 16)] = zv16

        @pl.loop(0, ACC_ROWS_PER_TILE // 64)
        def _zero(j):
            pltpu.sync_copy(
                z_v, acc.at[pl.ds(sid * ACC_ROWS_PER_TILE + j * 64, 64)])

        plsc.subcore_barrier()

        base = (cid * NSUB + sid) * CHUNKS_PER_TILE_H
        sems = (sem0, sem1)

        def load(c, slot, sem):
            pltpu.async_copy(m_hbm.at[pl.ds(c * CHUNK, CHUNK)],
                             m_v.at[slot], sem)
            pltpu.async_copy(d_hbm.at[pl.ds(c, 1)],
                             i_v.at[pl.ds(slot, 1)], sem)

        def drain_add(c, slot, sem):
            pltpu.make_async_copy(m_hbm.at[pl.ds(c * CHUNK, CHUNK)],
                                  m_v.at[slot], sem).wait()
            pltpu.make_async_copy(d_hbm.at[pl.ds(c, 1)],
                                  i_v.at[pl.ds(slot, 1)], sem).wait()
            pltpu.sync_copy(m_v.at[slot], acc.at[i_v.at[slot]], add=True)

        load(base, 0, sems[0])

        @pl.loop(0, CHUNKS_PER_TILE_H // 2)
        def _scat(t):
            c = base + 2 * t
            load(c + 1, 1, sems[1])
            drain_add(c, 0, sems[0])

            @pl.when(t < CHUNKS_PER_TILE_H // 2 - 1)
            def _():
                load(c + 2, 0, sems[0])

            drain_add(c + 1, 1, sems[1])

        plsc.subcore_barrier()

        @pl.loop(0, ACC_ROWS_PER_TILE // CHUNK)
        def _drain(j):
            r = sid * ACC_ROWS_PER_TILE + j * CHUNK
            pltpu.sync_copy(acc.at[pl.ds(r, CHUNK)],
                            o_hbm.at[cid, pl.ds(r, CHUNK)])

    return k(msg, dst2d)


def kernel(x, pos, edge_index, edge_attr, params):
    f32 = jnp.float32
    x = x.astype(jnp.int32)
    ei = edge_index.astype(jnp.int32)
    src, dst = ei[0], ei[1]
    padidx = jnp.full((EP - E,), N_NODES, jnp.int32)
    dst2d = jnp.concatenate([dst, padidx]).reshape(2, NCHUNKS_H, CHUNK)
    src2d = jnp.concatenate([src, padidx]).reshape(2, NCHUNKS_H, CHUNK)
    dst_h = (dst2d[0], dst2d[1])
    src_h = (src2d[0], src2d[1])

    def b2d(b):
        return b.reshape(1, -1)

    def pad_lane(a, n):
        return jnp.pad(a, ((0, 0), (0, n - a.shape[1])))

    oh = (x[:, None] == jnp.arange(9, dtype=jnp.int32)[None, :]).astype(f32)
    feat = jnp.concatenate([oh, jnp.zeros((N_NODES, 7), f32), pos], axis=1)
    feat = jnp.pad(feat, ((0, NP - N_NODES), (0, H - feat.shape[1])))

    pni = params["node_in"]
    (w1, b1), (w2, b2), (w3, b3) = pni["lin"]
    g, be = pni["ln"]
    emb_p = jnp.pad(params["embed"], ((0, 7), (0, 0)))
    w1a = w1[:16]
    w1b_p = jnp.pad(w1[16:30], ((0, (H - 16) - 14), (0, 0)))
    node = _node_encode(feat, emb_p, w1a, w1b_p, b2d(b1), w2, b2d(b2),
                        w3, b2d(b3), b2d(g), b2d(be))

    pei = params["edge_in"]
    (w1, b1), (w2, b2), (w3, b3) = pei["lin"]
    g, be = pei["ln"]
    ef = jnp.pad(edge_attr, ((0, EP - E), (0, H - edge_attr.shape[1])))
    w1_p = jnp.pad(w1, ((0, H - w1.shape[0]), (0, 0)))
    e_h = [_edge_encode_half(ef, hf, w1_p, b2d(b1), w2, b2d(b2), w3, b2d(b3),
                             b2d(g), b2d(be)) for hf in (0, 1)]


    for li, layer in enumerate(params["mp"]):
        (we1, eb1), (we2, eb2), (we3, eb3) = layer["lin_edge"]["lin"]
        eg, ebe = layer["lin_edge"]["ln"]
        ew = (we1, b2d(eb1), we2, b2d(eb2), we3, b2d(eb3), b2d(eg), b2d(ebe))
        g0d, g0s = _sc_gather(node, node, dst_h[0], src_h[0])
        g1d, g1s = _sc_gather(node, node, dst_h[1], src_h[1])
        e0, m0 = _edge_mlp(g0d, g0s, e_h[0], *ew)
        acc0 = _sc_scatter(m0, dst_h[0])
        e1, m1 = _edge_mlp(g1d, g1s, e_h[1], *ew)
        acc1 = _sc_scatter(m1, dst_h[1])
        e_h = [e0, e1]
        (wn1, nb1), (wn2, nb2), (wn3, nb3) = layer["lin_node"]["lin"]
        ng, nbe = layer["lin_node"]["ln"]
        node = _node_update(node, acc0[0], acc0[1], acc1[0], acc1[1],
                            wn1, b2d(nb1), wn2, b2d(nb2), wn3, b2d(nb3),
                            b2d(ng), b2d(nbe))

    pno = params["node_out"]
    (w1, b1), (w2, b2), (w3, b3) = pno["lin"]
    w3_p = pad_lane(w3, H)
    b3_p = jnp.pad(b3, (0, H - b3.shape[0]))
    out = _decode(node, w1, b2d(b1), w2, b2d(b2), w3_p, b2d(b3_p))
    return out[:N_NODES, :2]


# speedup vs baseline: 11.4639x; 1.3329x over previous
import functools

import jax
import jax.numpy as jnp
from jax import lax
from jax.experimental import pallas as pl
from jax.experimental.pallas import tpu as pltpu
from jax.experimental.pallas import tpu_sc as plsc

H = 128
N_NODES = 10000
NP = 10240
E = 320000
EP = 327680
EBLK = 1280
NBLK = 1024
NCORES = 2
NSUB = 16
CHUNK = 128
EPH = EP // 2
NCHUNKS_H = EPH // CHUNK
CHUNKS_PER_TILE_H = NCHUNKS_H // (NCORES * NSUB)
ACC_ROWS_PER_TILE = NP // NSUB

_PREC = lax.Precision.DEFAULT


def _dot(a, b):
    return lax.dot_general(a, b, (((1,), (0,)), ((), ())),
                           precision=_PREC, preferred_element_type=jnp.float32)


def _ln(x, g, b):
    mu = jnp.mean(x, axis=-1, keepdims=True)
    xc = x - mu
    var = jnp.mean(xc * xc, axis=-1, keepdims=True)
    return xc / jnp.sqrt(var + 1e-5) * g + b


def _full(shape):
    return pl.BlockSpec(shape, lambda i: tuple(0 for _ in shape))


def _row_spec(blk):
    return pl.BlockSpec((blk, H), lambda i: (i, 0))


def _node_enc_body(x_ref, emb_ref, w1a_ref, w1b_ref, b1_ref, w2_ref, b2_ref,
                   w3_ref, b3_ref, g_ref, be_ref, o_ref, ob_ref):
    w1_top = _dot(emb_ref[...], w1a_ref[...])
    x = x_ref[...]
    h_emb = lax.dot_general(x[:, :16], w1_top, (((1,), (0,)), ((), ())),
                            precision=lax.Precision.HIGHEST,
                            preferred_element_type=jnp.float32)
    h = jnp.maximum(h_emb + _dot(x[:, 16:], w1b_ref[...]) + b1_ref[...], 0.0)
    h = jnp.maximum(_dot(h, w2_ref[...]) + b2_ref[...], 0.0)
    h = _dot(h, w3_ref[...]) + b3_ref[...]
    o = _ln(h, g_ref[...], be_ref[...])
    o_ref[...] = o
    ob_ref[...] = o.astype(jnp.bfloat16)


def _mlp3_body(x_ref, w1_ref, b1_ref, w2_ref, b2_ref, w3_ref, b3_ref,
               g_ref, be_ref, o_ref):
    h = jnp.maximum(_dot(x_ref[...], w1_ref[...]) + b1_ref[...], 0.0)
    h = jnp.maximum(_dot(h, w2_ref[...]) + b2_ref[...], 0.0)
    h = _dot(h, w3_ref[...]) + b3_ref[...]
    o_ref[...] = _ln(h, g_ref[...], be_ref[...])


def _mlp3_noln_body(x_ref, w1_ref, b1_ref, w2_ref, b2_ref, w3_ref, b3_ref,
                    o_ref):
    h = jnp.maximum(_dot(x_ref[...], w1_ref[...]) + b1_ref[...], 0.0)
    h = jnp.maximum(_dot(h, w2_ref[...]) + b2_ref[...], 0.0)
    o_ref[...] = _dot(h, w3_ref[...]) + b3_ref[...]


def _ab_body(n_ref, wi_ref, wj_ref, a_ref, b_ref):
    a_ref[...] = _dot(n_ref[...], wi_ref[...])
    b_ref[...] = _dot(n_ref[...], wj_ref[...])


def _edge_body(gd_ref, gs_ref, e_ref, w1e_ref, b1_ref, w2_ref, b2_ref,
               w3_ref, b3_ref, g_ref, be_ref, eo_ref, mo_ref):
    x = e_ref[...]
    h = gd_ref[...] + gs_ref[...] + _dot(x, w1e_ref[...]) + b1_ref[...]
    h = jnp.maximum(h, 0.0)
    h = jnp.maximum(_dot(h, w2_ref[...]) + b2_ref[...], 0.0)
    m = _ln(_dot(h, w3_ref[...]) + b3_ref[...], g_ref[...], be_ref[...])
    mo_ref[...] = m
    eo_ref[...] = x + m


def _node_upd_body(n_ref, a0_ref, a1_ref, a2_ref, a3_ref, wn_ref, wa_ref,
                   b1_ref, w2_ref, b2_ref, w3_ref, b3_ref, g_ref, be_ref,
                   no_ref, nb_ref):
    x = n_ref[...]
    acc = (a0_ref[...] + a1_ref[...]) + (a2_ref[...] + a3_ref[...])
    h = jnp.maximum(_dot(x, wn_ref[...]) + _dot(acc, wa_ref[...]) + b1_ref[...], 0.0)
    h = jnp.maximum(_dot(h, w2_ref[...]) + b2_ref[...], 0.0)
    o = x + _ln(_dot(h, w3_ref[...]) + b3_ref[...], g_ref[...], be_ref[...])
    no_ref[...] = o
    nb_ref[...] = o.astype(jnp.bfloat16)


def _wspecs(shapes):
    return [_full(s) for s in shapes]


def _node_encode(feat, emb_p, w1a, w1b_p, b1, w2, b2, w3, b3, g, be):
    return pl.pallas_call(
        _node_enc_body,
        grid=(NP // NBLK,),
        in_specs=[_row_spec(NBLK)] + _wspecs([
            (16, 16), (16, H), (H - 16, H), (1, H), (H, H), (1, H),
            (H, H), (1, H), (1, H), (1, H)]),
        out_specs=[_row_spec(NBLK), _row_spec(NBLK)],
        out_shape=[jax.ShapeDtypeStruct((NP, H), jnp.float32),
                   jax.ShapeDtypeStruct((NP, H), jnp.bfloat16)],
    )(feat, emb_p, w1a, w1b_p, b1, w2, b2, w3, b3, g, be)


def _edge_encode_half(feat, half, w1, b1, w2, b2, w3, b3, g, be):
    nblocks = EPH // EBLK
    return pl.pallas_call(
        _mlp3_body,
        grid=(nblocks,),
        in_specs=[pl.BlockSpec((EBLK, H), lambda i: (i + half * nblocks, 0))]
        + _wspecs([
            (H, H), (1, H), (H, H), (1, H), (H, H), (1, H), (1, H), (1, H)]),
        out_specs=_row_spec(EBLK),
        out_shape=jax.ShapeDtypeStruct((EPH, H), jnp.float32),
    )(feat, w1, b1, w2, b2, w3, b3, g, be)


def _ab_project(node, wi, wj):
    return pl.pallas_call(
        _ab_body,
        grid=(NP // NBLK,),
        in_specs=[_row_spec(NBLK)] + _wspecs([(H, H), (H, H)]),
        out_specs=[_row_spec(NBLK), _row_spec(NBLK)],
        out_shape=[jax.ShapeDtypeStruct((NP, H), jnp.float32)] * 2,
    )(node, wi, wj)


def _edge_mlp(gd, gs, e, w1e, b1, w2, b2, w3, b3, g, be):
    return pl.pallas_call(
        _edge_body,
        grid=(EPH // EBLK,),
        in_specs=[_row_spec(EBLK)] * 3 + _wspecs([
            (H, H), (1, H), (H, H), (1, H), (H, H), (1, H), (1, H), (1, H)]),
        out_specs=[_row_spec(EBLK), _row_spec(EBLK)],
        out_shape=[jax.ShapeDtypeStruct((EPH, H), jnp.float32)] * 2,
    )(gd, gs, e, w1e, b1, w2, b2, w3, b3, g, be)


def _node_update(node, acc0, acc1, acc2, acc3, wn, wa, b1, w2, b2, w3, b3,
                 g, be):
    return pl.pallas_call(
        _node_upd_body,
        grid=(NP // NBLK,),
        in_specs=[_row_spec(NBLK)] * 5 + _wspecs([
            (H, H), (H, H), (1, H), (H, H), (1, H), (H, H), (1, H),
            (1, H), (1, H)]),
        out_specs=[_row_spec(NBLK), _row_spec(NBLK)],
        out_shape=[jax.ShapeDtypeStruct((NP, H), jnp.float32),
                   jax.ShapeDtypeStruct((NP, H), jnp.bfloat16)],
    )(node, acc0, acc1, acc2, acc3, wn, wa, b1, w2, b2, w3, b3, g, be)


def _decode(node, w1, b1, w2, b2, w3, b3):
    return pl.pallas_call(
        _mlp3_noln_body,
        grid=(NP // NBLK,),
        in_specs=[_row_spec(NBLK)] + _wspecs([
            (H, H), (1, H), (H, H), (1, H), (H, H), (1, H)]),
        out_specs=_row_spec(NBLK),
        out_shape=jax.ShapeDtypeStruct((NP, H), jnp.float32),
    )(node, w1, b1, w2, b2, w3, b3)


def _sc_mesh():
    return plsc.VectorSubcoreMesh(core_axis_name="core",
                                  subcore_axis_name="subcore")


def _sc_gather(a, b, dst2d, src2d):
    out_t = (jax.ShapeDtypeStruct((EPH, H), jnp.float32),
             jax.ShapeDtypeStruct((EPH, H), jnp.float32))

    @functools.partial(pl.kernel, out_type=out_t, mesh=_sc_mesh(),
                       scratch_types=[pltpu.SemaphoreType.DMA,
                                      pltpu.SemaphoreType.DMA])
    def k(a_hbm, b_hbm, d_hbm, s_hbm, gd_hbm, gs_hbm, sg0, sg1):
        def body(d_v, s_v, gd_v, gs_v):
            c0 = pltpu.async_copy(a_hbm.at[d_v.at[0]], gd_v, sg0)
            c1 = pltpu.async_copy(b_hbm.at[s_v.at[0]], gs_v, sg1)
            c0.wait()
            c1.wait()

        pltpu.emit_pipeline(
            body,
            grid=(NCHUNKS_H,),
            in_specs=[pl.BlockSpec((1, CHUNK), lambda i: (i, 0)),
                      pl.BlockSpec((1, CHUNK), lambda i: (i, 0))],
            out_specs=[pl.BlockSpec((CHUNK, H), lambda i: (i, 0)),
                       pl.BlockSpec((CHUNK, H), lambda i: (i, 0))],
            core_axis_name=("core", "subcore"),
            dimension_semantics=(pltpu.PARALLEL,),
        )(d_hbm, s_hbm, gd_hbm, gs_hbm)

    return k(a, b, dst2d, src2d)


def _sc_scatter(msg, dst2d, zeros):
    @functools.partial(
        pl.kernel,
        out_type=jax.ShapeDtypeStruct((NCORES, NP, H), jnp.float32),
        mesh=_sc_mesh(),
        scratch_types=[
            pltpu.VMEM((2, CHUNK, H), jnp.float32),
            pltpu.VMEM((2, CHUNK), jnp.int32),
            pltpu.VMEM_SHARED((NP, H), jnp.float32),
            pltpu.SemaphoreType.DMA,
            pltpu.SemaphoreType.DMA,
        ],
    )
    def k(m_hbm, d_hbm, z_hbm, o_hbm, m_v, i_v, acc, sem0, sem1):
        cid = lax.axis_index("core")
        sid = lax.axis_index("subcore")

        @pl.loop(0, ACC_ROWS_PER_TILE // CHUNK)
        def _zero(j):
            pltpu.sync_copy(
                z_hbm, acc.at[pl.ds(sid * ACC_ROWS_PER_TILE + j * CHUNK, CHUNK)])

        plsc.subcore_barrier()

        base = (cid * NSUB + sid) * CHUNKS_PER_TILE_H
        sems = (sem0, sem1)

        def load(c, slot, sem):
            pltpu.async_copy(m_hbm.at[pl.ds(c * CHUNK, CHUNK)],
                             m_v.at[slot], sem)
            pltpu.async_copy(d_hbm.at[pl.ds(c, 1)],
                             i_v.at[pl.ds(slot, 1)], sem)

        def drain_add(c, slot, sem):
            pltpu.make_async_copy(m_hbm.at[pl.ds(c * CHUNK, CHUNK)],
                                  m_v.at[slot], sem).wait()
            pltpu.make_async_copy(d_hbm.at[pl.ds(c, 1)],
                                  i_v.at[pl.ds(slot, 1)], sem).wait()
            pltpu.sync_copy(m_v.at[slot], acc.at[i_v.at[slot]], add=True)

        load(base, 0, sems[0])

        @pl.loop(0, CHUNKS_PER_TILE_H // 2)
        def _scat(t):
            c = base + 2 * t
            load(c + 1, 1, sems[1])
            drain_add(c, 0, sems[0])

            @pl.when(t < CHUNKS_PER_TILE_H // 2 - 1)
            def _():
                load(c + 2, 0, sems[0])

            drain_add(c + 1, 1, sems[1])

        plsc.subcore_barrier()

        @pl.loop(0, ACC_ROWS_PER_TILE // CHUNK)
        def _drain(j):
            r = sid * ACC_ROWS_PER_TILE + j * CHUNK
            pltpu.sync_copy(acc.at[pl.ds(r, CHUNK)],
                            o_hbm.at[cid, pl.ds(r, CHUNK)])

    return k(msg, dst2d, zeros)


def kernel(x, pos, edge_index, edge_attr, params):
    f32 = jnp.float32
    x = x.astype(jnp.int32)
    ei = edge_index.astype(jnp.int32)
    src, dst = ei[0], ei[1]
    padidx = jnp.full((EP - E,), N_NODES, jnp.int32)
    dst2d = jnp.concatenate([dst, padidx]).reshape(2, NCHUNKS_H, CHUNK)
    src2d = jnp.concatenate([src, padidx]).reshape(2, NCHUNKS_H, CHUNK)
    dst_h = (dst2d[0], dst2d[1])
    src_h = (src2d[0], src2d[1])

    def b2d(b):
        return b.reshape(1, -1)

    def pad_lane(a, n):
        return jnp.pad(a, ((0, 0), (0, n - a.shape[1])))

    oh = (x[:, None] == jnp.arange(9, dtype=jnp.int32)[None, :]).astype(f32)
    feat = jnp.concatenate([oh, jnp.zeros((N_NODES, 7), f32), pos], axis=1)
    feat = jnp.pad(feat, ((0, NP - N_NODES), (0, H - feat.shape[1])))

    pni = params["node_in"]
    (w1, b1), (w2, b2), (w3, b3) = pni["lin"]
    g, be = pni["ln"]
    emb_p = jnp.pad(params["embed"], ((0, 7), (0, 0)))
    w1a = w1[:16]
    w1b_p = jnp.pad(w1[16:30], ((0, (H - 16) - 14), (0, 0)))
    node, node_bf = _node_encode(feat, emb_p, w1a, w1b_p, b2d(b1), w2, b2d(b2),
                                 w3, b2d(b3), b2d(g), b2d(be))

    pei = params["edge_in"]
    (w1, b1), (w2, b2), (w3, b3) = pei["lin"]
    g, be = pei["ln"]
    ef = jnp.pad(edge_attr, ((0, EP - E), (0, H - edge_attr.shape[1])))
    w1_p = jnp.pad(w1, ((0, H - w1.shape[0]), (0, 0)))
    e_h = [_edge_encode_half(ef, hf, w1_p, b2d(b1), w2, b2d(b2), w3, b2d(b3),
                             b2d(g), b2d(be)) for hf in (0, 1)]

    zeros = jnp.zeros((CHUNK, H), f32)

    for layer in params["mp"]:
        (we1, eb1), (we2, eb2), (we3, eb3) = layer["lin_edge"]["lin"]
        eg, ebe = layer["lin_edge"]["ln"]
        w1i, w1j, w1e = we1[:H], we1[H:2 * H], we1[2 * H:]
        a, b = _ab_project(node, w1i, w1j)
        ew = (w1e, b2d(eb1), we2, b2d(eb2), we3, b2d(eb3), b2d(eg), b2d(ebe))
        g0d, g0s = _sc_gather(a, b, dst_h[0], src_h[0])
        g1d, g1s = _sc_gather(a, b, dst_h[1], src_h[1])
        e0, m0 = _edge_mlp(g0d, g0s, e_h[0], *ew)
        acc0 = _sc_scatter(m0, dst_h[0], zeros)
        e1, m1 = _edge_mlp(g1d, g1s, e_h[1], *ew)
        acc1 = _sc_scatter(m1, dst_h[1], zeros)
        e_h = [e0, e1]
        (wn1, nb1), (wn2, nb2), (wn3, nb3) = layer["lin_node"]["lin"]
        ng, nbe = layer["lin_node"]["ln"]
        node, node_bf = _node_update(node, acc0[0], acc0[1], acc1[0], acc1[1],
                                     wn1[:H], wn1[H:],
                                     b2d(nb1), wn2, b2d(nb2), wn3, b2d(nb3),
                                     b2d(ng), b2d(nbe))

    pno = params["node_out"]
    (w1, b1), (w2, b2), (w3, b3) = pno["lin"]
    w3_p = pad_lane(w3, H)
    b3_p = jnp.pad(b3, (0, H - b3.shape[0]))
    out = _decode(node, w1, b2d(b1), w2, b2d(b2), w3_p, b2d(b3_p))
    return out[:N_NODES, :2]
